# baseline (speedup 1.0000x reference)
import jax
import jax.numpy as jnp
from jax.experimental import pallas as pl
from jax.experimental.pallas import tpu as pltpu

_RCHUNKS = 8


def _moe_kernel(x_ref, w1_ref, w2_ref, w3_ref, w4_ref, b_ref, wg_ref, out_ref,
                wbf_ref, wf_ref, zb_ref, lg_ref, sem):
    i = pl.program_id(0)
    h = out_ref.shape[1]
    d_in = x_ref.shape[1]
    rc = d_in // _RCHUNKS

    @pl.when(i == 0)
    def _load_w():
        w_hbm = [w1_ref, w2_ref, w3_ref, w4_ref]
        n_chunks = 4 * _RCHUNKS

        def desc(idx):
            e, r = divmod(idx, _RCHUNKS)
            buf = idx % 2
            return e, r, pltpu.make_async_copy(
                w_hbm[e].at[pl.ds(r * rc, rc), :], wf_ref.at[buf], sem.at[buf])

        _, _, first = desc(0)
        first.start()
        for idx in range(n_chunks):
            if idx + 1 < n_chunks:
                _, _, nxt = desc(idx + 1)
                nxt.start()
            e, r, cur = desc(idx)
            cur.wait()
            wbf_ref[pl.ds(r * rc, rc), pl.ds(e * h, h)] = (
                wf_ref[idx % 2].astype(jnp.bfloat16))

    nsteps = pl.num_programs(0) - 1

    @pl.when(i < nsteps)
    def _dot():
        xb = x_ref[:].astype(jnp.bfloat16)
        z = jnp.dot(xb, wbf_ref[:], preferred_element_type=jnp.float32)
        z = jnp.maximum(z + b_ref[0][None, :], 0.0)
        zb = z.astype(jnp.bfloat16)
        zb_ref[i % 2] = zb
        lg_ref[i % 2] = jnp.dot(zb, wg_ref[:], preferred_element_type=jnp.float32)

    @pl.when(i > 0)
    def _epilogue():
        j = (i - 1) % 2
        zb = zb_ref[j]
        gates = jax.nn.softmax(lg_ref[j], axis=1)
        acc = gates[:, 0:1] * zb[:, 0:h].astype(jnp.float32)
        for e in range(1, 4):
            acc = acc + gates[:, e:e + 1] * zb[:, e * h:(e + 1) * h].astype(jnp.float32)
        out_ref[:] = acc


def kernel(x, W1, b1, W2, b2, W3, b3, W4, b4, w_gate):
    n, d_in = x.shape
    h = W1.shape[1]
    bc = jnp.concatenate([b1, b2, b3, b4]).reshape(1, 4 * h)
    tm = 256
    nsteps = n // tm
    grid = (nsteps + 1,)
    wspec = pl.BlockSpec(memory_space=pltpu.HBM)
    return pl.pallas_call(
        _moe_kernel,
        grid=grid,
        in_specs=[
            pl.BlockSpec((tm, d_in), lambda i: (jnp.minimum(i, n // 256 - 1), 0)),
            wspec, wspec, wspec, wspec,
            pl.BlockSpec((1, 4 * h), lambda i: (0, 0)),
            pl.BlockSpec((4 * h, 4), lambda i: (0, 0)),
        ],
        out_specs=pl.BlockSpec((tm, h), lambda i: (jnp.maximum(i - 1, 0), 0)),
        out_shape=jax.ShapeDtypeStruct((n, h), jnp.float32),
        scratch_shapes=[pltpu.VMEM((d_in, 4 * h), jnp.bfloat16),
                        pltpu.VMEM((2, d_in // _RCHUNKS, h), jnp.float32),
                        pltpu.VMEM((2, tm, 4 * h), jnp.bfloat16),
                        pltpu.VMEM((2, tm, 4), jnp.float32),
                        pltpu.SemaphoreType.DMA((2,))],
        compiler_params=pltpu.CompilerParams(
            dimension_semantics=("arbitrary",),
        ),
    )(x, W1, W2, W3, W4, bc, w_gate.astype(jnp.bfloat16))


# speedup vs baseline: 1.0130x; 1.0130x over previous
import jax
import jax.numpy as jnp
from jax.experimental import pallas as pl
from jax.experimental.pallas import tpu as pltpu

_RCHUNKS = 8


def _gate_combine(zb, lg, h, out_ref):
    gates = jax.nn.softmax(lg, axis=1)
    acc = gates[:, 0:1] * zb[:, 0:h].astype(jnp.float32)
    for e in range(1, 4):
        acc = acc + gates[:, e:e + 1] * zb[:, e * h:(e + 1) * h].astype(jnp.float32)
    out_ref[:] = acc


def _moe_kernel(x_ref, w1_ref, w2_ref, w3_ref, w4_ref, b_ref, wg_ref, out_ref,
                wbf_ref, wf_ref, zb_ref, sem):
    i = pl.program_id(0)
    h = out_ref.shape[1]
    d_in = x_ref.shape[1]
    rc = d_in // _RCHUNKS

    @pl.when(i == 0)
    def _load_w_and_compute():
        w_hbm = [w1_ref, w2_ref, w3_ref, w4_ref]
        n_chunks = 4 * _RCHUNKS

        def desc(idx):
            e, r = divmod(idx, _RCHUNKS)
            buf = idx % 2
            return e, r, pltpu.make_async_copy(
                w_hbm[e].at[pl.ds(r * rc, rc), :], wf_ref.at[buf], sem.at[buf])

        xb = x_ref[:].astype(jnp.bfloat16)
        _, _, first = desc(0)
        first.start()
        lg = None
        for idx in range(n_chunks):
            if idx + 1 < n_chunks:
                _, _, nxt = desc(idx + 1)
                nxt.start()
            e, r, cur = desc(idx)
            cur.wait()
            wbf_ref[pl.ds(r * rc, rc), pl.ds(e * h, h)] = (
                wf_ref[idx % 2].astype(jnp.bfloat16))
            if r == _RCHUNKS - 1:
                z = jnp.dot(xb, wbf_ref[:, pl.ds(e * h, h)],
                            preferred_element_type=jnp.float32)
                z = jnp.maximum(z + b_ref[0, pl.ds(e * h, h)][None, :], 0.0)
                zbe = z.astype(jnp.bfloat16)
                zb_ref[:, pl.ds(e * h, h)] = zbe
                lge = jnp.dot(zbe, wg_ref[pl.ds(e * h, h), :],
                              preferred_element_type=jnp.float32)
                lg = lge if lg is None else lg + lge
        _gate_combine(zb_ref[:], lg, h, out_ref)

    @pl.when(i > 0)
    def _body():
        xb = x_ref[:].astype(jnp.bfloat16)
        z = jnp.dot(xb, wbf_ref[:], preferred_element_type=jnp.float32)
        z = jnp.maximum(z + b_ref[0][None, :], 0.0)
        zb = z.astype(jnp.bfloat16)
        lg = jnp.dot(zb, wg_ref[:], preferred_element_type=jnp.float32)
        _gate_combine(zb, lg, h, out_ref)


def kernel(x, W1, b1, W2, b2, W3, b3, W4, b4, w_gate):
    n, d_in = x.shape
    h = W1.shape[1]
    bc = jnp.concatenate([b1, b2, b3, b4]).reshape(1, 4 * h)
    tm = 256
    grid = (n // tm,)
    wspec = pl.BlockSpec(memory_space=pltpu.HBM)
    return pl.pallas_call(
        _moe_kernel,
        grid=grid,
        in_specs=[
            pl.BlockSpec((tm, d_in), lambda i: (i, 0)),
            wspec, wspec, wspec, wspec,
            pl.BlockSpec((1, 4 * h), lambda i: (0, 0)),
            pl.BlockSpec((4 * h, 4), lambda i: (0, 0)),
        ],
        out_specs=pl.BlockSpec((tm, h), lambda i: (i, 0)),
        out_shape=jax.ShapeDtypeStruct((n, h), jnp.float32),
        scratch_shapes=[pltpu.VMEM((d_in, 4 * h), jnp.bfloat16),
                        pltpu.VMEM((2, d_in // _RCHUNKS, h), jnp.float32),
                        pltpu.VMEM((tm, 4 * h), jnp.bfloat16),
                        pltpu.SemaphoreType.DMA((2,))],
        compiler_params=pltpu.CompilerParams(
            dimension_semantics=("arbitrary",),
        ),
    )(x, W1, W2, W3, W4, bc, w_gate.astype(jnp.bfloat16))
